# baseline (speedup 1.0000x reference)
import functools

import jax
import jax.numpy as jnp
from jax import lax
from jax.experimental import pallas as pl
from jax.experimental.pallas import tpu as pltpu
from jax.experimental.pallas import tpu_sc as plsc

_BATCH = 16384
_EMB = 64
_NC = 2
_NS = 16
_NW = _NC * _NS
_BPW = _BATCH // _NW
_CH = 128
_NCHUNK = _BPW // _CH
_GPC = _CH // 16

_ROWS = 1000000
_TC_N = 7813
_TR_STRIDE = _TC_N * 1024


def _rsqrt_newton(x):
    i = lax.bitcast_convert_type(x, jnp.int32)
    i = jnp.int32(0x5F3759DF) - lax.shift_right_arithmetic(i, jnp.int32(1))
    y = lax.bitcast_convert_type(i, jnp.float32)
    half, three_half = jnp.float32(0.5), jnp.float32(1.5)
    for _ in range(3):
        y = y * (three_half - half * x * y * y)
    return y


def _phys_base(ev):
    return ev


_DIM_OFF = [b * _ROWS for b in range(_EMB)]


def _sc_body(hi_hbm, ri_hbm, ti_hbm, ent_hbm, rel_hbm, out_hbm,
             hi_v, ri_v, ti_v, hx_v, rx_v, tx_v, h_v, r_v, t_v, o_v,
             sem0, sem1, sem2):
    wid = lax.axis_index("s") * _NC + lax.axis_index("c")
    base = wid * _BPW

    pltpu.sync_copy(hi_hbm.at[pl.ds(base, _BPW)], hi_v)
    pltpu.sync_copy(ri_hbm.at[pl.ds(base, _BPW)], ri_v)
    pltpu.sync_copy(ti_hbm.at[pl.ds(base, _BPW)], ti_v)

    eps = jnp.float32(1e-24)

    def build_indices(c, iv, scr):
        def grp(g, carry):
            ev = iv[pl.ds(c * _CH + g * 16, 16)]
            pb = _phys_base(ev)
            for b in range(_EMB):
                scr[pl.ds(b * _CH + g * 16, 16)] = pb + jnp.int32(_DIM_OFF[b])
            return carry

        lax.fori_loop(0, _GPC, grp, 0)

    def fire(c):
        build_indices(c, hi_v, hx_v)
        build_indices(c, ri_v, rx_v)
        build_indices(c, ti_v, tx_v)
        cph = pltpu.async_copy(ent_hbm.at[hx_v], h_v, sem0)
        cpr = pltpu.async_copy(rel_hbm.at[rx_v], r_v, sem1)
        cpt = pltpu.async_copy(ent_hbm.at[tx_v], t_v, sem2)
        return cph, cpr, cpt

    def compute(c):
        def grp(g, carry):
            g16 = g * 16
            ssh = jnp.full((16,), eps, jnp.float32)
            ssr = jnp.full((16,), eps, jnp.float32)
            sst = jnp.full((16,), eps, jnp.float32)
            for b in range(_EMB):
                hd = h_v[pl.ds(b * _CH + g16, 16)]
                rd = r_v[pl.ds(b * _CH + g16, 16)]
                td = t_v[pl.ds(b * _CH + g16, 16)]
                ssh = ssh + hd * hd
                ssr = ssr + rd * rd
                sst = sst + td * td
            ih = _rsqrt_newton(ssh)
            ir = _rsqrt_newton(ssr)
            it = _rsqrt_newton(sst)
            acc = jnp.zeros((16,), jnp.float32)
            for b in range(_EMB):
                hd = h_v[pl.ds(b * _CH + g16, 16)]
                rd = r_v[pl.ds(b * _CH + g16, 16)]
                td = t_v[pl.ds(b * _CH + g16, 16)]
                acc = acc + jnp.abs(hd * ih + rd * ir - td * it)
            o_v[pl.ds(c * _CH + g16, 16)] = acc
            return carry

        lax.fori_loop(0, _GPC, grp, 0)

    for c in range(_NCHUNK):
        cph, cpr, cpt = fire(c)
        cph.wait()
        cpr.wait()
        cpt.wait()
        compute(c)

    pltpu.sync_copy(o_v, out_hbm.at[pl.ds(base, _BPW)])


@functools.partial(
    pl.kernel,
    out_type=jax.ShapeDtypeStruct((_BATCH,), jnp.float32),
    mesh=plsc.VectorSubcoreMesh(core_axis_name="c", subcore_axis_name="s"),
    compiler_params=pltpu.CompilerParams(needs_layout_passes=False),
    scratch_types=[
        pltpu.VMEM((_BPW,), jnp.int32),
        pltpu.VMEM((_BPW,), jnp.int32),
        pltpu.VMEM((_BPW,), jnp.int32),
        pltpu.VMEM((_EMB * _CH,), jnp.int32),
        pltpu.VMEM((_EMB * _CH,), jnp.int32),
        pltpu.VMEM((_EMB * _CH,), jnp.int32),
        pltpu.VMEM((_EMB * _CH,), jnp.float32),
        pltpu.VMEM((_EMB * _CH,), jnp.float32),
        pltpu.VMEM((_EMB * _CH,), jnp.float32),
        pltpu.VMEM((_BPW,), jnp.float32),
        pltpu.SemaphoreType.DMA,
        pltpu.SemaphoreType.DMA,
        pltpu.SemaphoreType.DMA,
    ],
)
def _transe_sc(*refs):
    _sc_body(*refs)


def _phys_flat(table):
    return table.T.reshape(-1)


def kernel(edge, entity_embedding, relation_embedding):
    h_idx = edge[:, 0]
    r_idx = edge[:, 1]
    t_idx = edge[:, 2]
    return _transe_sc(h_idx, r_idx, t_idx,
                      _phys_flat(entity_embedding),
                      _phys_flat(relation_embedding))


# speedup vs baseline: 8.7545x; 8.7545x over previous
import functools

import jax
import jax.numpy as jnp
from jax import lax
from jax.experimental import pallas as pl
from jax.experimental.pallas import tpu as pltpu
from jax.experimental.pallas import tpu_sc as plsc

_BATCH = 16384
_EMB = 64
_NC = 2
_NS = 16
_NW = _NC * _NS
_BPW = _BATCH // _NW
_CH = 256
_NCHUNK = _BPW // _CH
_GROUPS = _CH // 16


def _rsqrt_newton(x):
    i = lax.bitcast_convert_type(x, jnp.int32)
    i = jnp.int32(0x5F3759DF) - lax.shift_right_arithmetic(i, jnp.int32(1))
    y = lax.bitcast_convert_type(i, jnp.float32)
    half, three_half = jnp.float32(0.5), jnp.float32(1.5)
    for _ in range(3):
        y = y * (three_half - half * x * y * y)
    return y


def _sc_body(hi_hbm, ri_hbm, ti_hbm, ent_hbm, rel_hbm, out_hbm,
             hi_v, ri_v, ti_v, hp_v, rp_v, tp_v, h_v, r_v, t_v,
             o_v, nrm_v, sem0, sem1, sem2):
    wid = lax.axis_index("s") * _NC + lax.axis_index("c")
    base = wid * _BPW

    pltpu.sync_copy(hi_hbm.at[pl.ds(base, _BPW)], hi_v)
    pltpu.sync_copy(ri_hbm.at[pl.ds(base, _BPW)], ri_v)
    pltpu.sync_copy(ti_hbm.at[pl.ds(base, _BPW)], ti_v)

    eps = jnp.float32(1e-24)
    zeros = jnp.zeros((16,), jnp.float32)
    ones_i = jnp.full((16,), 1, jnp.int32)
    c1 = jnp.full((16,), _CH, jnp.int32)
    c2 = jnp.full((16,), 2 * _CH, jnp.int32)

    def pair_idx(c, carry):
        s = pl.ds(c * _CH + carry * 16, 16)
        d = pl.ds(carry * 16, 16)
        hp_v[d] = lax.shift_right_logical(hi_v[s], jnp.int32(1))
        rp_v[d] = lax.shift_right_logical(ri_v[s], jnp.int32(1))
        tp_v[d] = lax.shift_right_logical(ti_v[s], jnp.int32(1))
        return carry + 1

    def chunk(c):
        lax.fori_loop(0, _GROUPS, lambda g, _: pair_idx(c, g), 0)
        cph = pltpu.async_copy(ent_hbm.at[hp_v], h_v, sem0)
        cpr = pltpu.async_copy(rel_hbm.at[rp_v], r_v, sem1)
        cpt = pltpu.async_copy(ent_hbm.at[tp_v], t_v, sem2)
        cph.wait()
        cpr.wait()
        cpt.wait()

        def zero_nrm(b, carry):
            nrm_v[pl.ds(b * 16, 16)] = zeros
            return carry

        lax.fori_loop(0, _GROUPS * 3, zero_nrm, 0)

        def ssq_group(g, carry):
            hv16 = hi_v[pl.ds(c * _CH + g * 16, 16)]
            rv16 = ri_v[pl.ds(c * _CH + g * 16, 16)]
            tv16 = ti_v[pl.ds(c * _CH + g * 16, 16)]
            jv = jnp.full((16,), g * 16, jnp.int32)
            for j in range(16):
                i = g * 16 + j
                ho = (hv16[j] & 1) * 64
                ro = (rv16[j] & 1) * 64
                to = (tv16[j] & 1) * 64
                hc = [h_v[i, pl.ds(ho + k * 16, 16)] for k in range(4)]
                rc = [r_v[i, pl.ds(ro + k * 16, 16)] for k in range(4)]
                tc = [t_v[i, pl.ds(to + k * 16, 16)] for k in range(4)]
                sh = hc[0] * hc[0] + hc[1] * hc[1] + hc[2] * hc[2] + hc[3] * hc[3]
                sr = rc[0] * rc[0] + rc[1] * rc[1] + rc[2] * rc[2] + rc[3] * rc[3]
                st = tc[0] * tc[0] + tc[1] * tc[1] + tc[2] * tc[2] + tc[3] * tc[3]
                plsc.addupdate_scatter(nrm_v, [jv], sh)
                plsc.addupdate_scatter(nrm_v, [jv + c1], sr)
                plsc.addupdate_scatter(nrm_v, [jv + c2], st)
                jv = jv + ones_i
            return carry

        lax.fori_loop(0, _GROUPS, ssq_group, 0)

        def newton16(b, carry):
            nrm_v[pl.ds(b * 16, 16)] = _rsqrt_newton(
                jnp.maximum(nrm_v[pl.ds(b * 16, 16)], eps))
            return carry

        lax.fori_loop(0, _GROUPS * 3, newton16, 0)

        def zero_out(b, carry):
            o_v[pl.ds(c * _CH + b * 16, 16)] = zeros
            return carry

        lax.fori_loop(0, _GROUPS, zero_out, 0)

        def dist_group(g, carry):
            hv16 = hi_v[pl.ds(c * _CH + g * 16, 16)]
            rv16 = ri_v[pl.ds(c * _CH + g * 16, 16)]
            tv16 = ti_v[pl.ds(c * _CH + g * 16, 16)]
            jv = jnp.full((16,), g * 16, jnp.int32)
            ov = jnp.full((16,), c * _CH + g * 16, jnp.int32)
            for j in range(16):
                i = g * 16 + j
                ho = (hv16[j] & 1) * 64
                ro = (rv16[j] & 1) * 64
                to = (tv16[j] & 1) * 64
                hc = [h_v[i, pl.ds(ho + k * 16, 16)] for k in range(4)]
                rc = [r_v[i, pl.ds(ro + k * 16, 16)] for k in range(4)]
                tc = [t_v[i, pl.ds(to + k * 16, 16)] for k in range(4)]
                ih = plsc.load_gather(nrm_v, [jv])
                ir = plsc.load_gather(nrm_v, [jv + c1])
                it = plsc.load_gather(nrm_v, [jv + c2])
                s = jnp.abs(hc[0] * ih + rc[0] * ir - tc[0] * it)
                for k in range(1, 4):
                    s = s + jnp.abs(hc[k] * ih + rc[k] * ir - tc[k] * it)
                plsc.addupdate_scatter(o_v, [ov], s)
                jv = jv + ones_i
                ov = ov + ones_i
            return carry

        lax.fori_loop(0, _GROUPS, dist_group, 0)

    for c in range(_NCHUNK):
        chunk(c)

    pltpu.sync_copy(o_v, out_hbm.at[pl.ds(base, _BPW)])


@functools.partial(
    pl.kernel,
    out_type=jax.ShapeDtypeStruct((_BATCH,), jnp.float32),
    mesh=plsc.VectorSubcoreMesh(core_axis_name="c", subcore_axis_name="s"),
    compiler_params=pltpu.CompilerParams(needs_layout_passes=False),
    scratch_types=[
        pltpu.VMEM((_BPW,), jnp.int32),
        pltpu.VMEM((_BPW,), jnp.int32),
        pltpu.VMEM((_BPW,), jnp.int32),
        pltpu.VMEM((_CH,), jnp.int32),
        pltpu.VMEM((_CH,), jnp.int32),
        pltpu.VMEM((_CH,), jnp.int32),
        pltpu.VMEM((_CH, 128), jnp.float32),
        pltpu.VMEM((_CH, 128), jnp.float32),
        pltpu.VMEM((_CH, 128), jnp.float32),
        pltpu.VMEM((_BPW,), jnp.float32),
        pltpu.VMEM((3 * _CH,), jnp.float32),
        pltpu.SemaphoreType.DMA,
        pltpu.SemaphoreType.DMA,
        pltpu.SemaphoreType.DMA,
    ],
)
def _transe_sc(*refs):
    _sc_body(*refs)


def kernel(edge, entity_embedding, relation_embedding):
    h_idx = edge[:, 0]
    r_idx = edge[:, 1]
    t_idx = edge[:, 2]
    ent2 = entity_embedding.reshape(500000, 128)
    rel2 = relation_embedding.reshape(500000, 128)
    return _transe_sc(h_idx, r_idx, t_idx, ent2, rel2)


# speedup vs baseline: 22.6047x; 2.5821x over previous
import functools

import jax
import jax.numpy as jnp
from jax import lax
from jax.experimental import pallas as pl
from jax.experimental.pallas import tpu as pltpu
from jax.experimental.pallas import tpu_sc as plsc

_BATCH = 16384
_EMB = 64
_NC = 2
_NS = 16
_NW = _NC * _NS
_BPW = _BATCH // _NW
_GROUPS = _BPW // 16
_ROWS = 1000000
_ALIGNED_TOP = 999424
_RW = 32768
_WIN = 1024
_ENT_CAP = 4096
_REL_CAP = 4096
_WW_CAP = 512
_PITCH = 72
_HT_TRASH = 2 * _BATCH
_R_TRASH = _BATCH


def _rsqrt_newton(x):
    i = lax.bitcast_convert_type(x, jnp.int32)
    i = jnp.int32(0x5F3759DF) - lax.shift_right_arithmetic(i, jnp.int32(1))
    y = lax.bitcast_convert_type(i, jnp.float32)
    half, three_half = jnp.float32(0.5), jnp.float32(1.5)
    for _ in range(3):
        y = y * (three_half - half * x * y * y)
    return y


def _scan_body(hi_hbm, ri_hbm, ti_hbm, ent_hbm, rel_hbm,
               etail_hbm, rtail_hbm, ht_out, r_out,
               idb_v, wli_v, wls_v, wwi_v, wws_v, wb_v, tt_v, ex_v,
               semw, semo):
    wid = lax.axis_index("s") * _NC + lax.axis_index("c")
    lo = wid * _RW
    hib = lo + _RW
    lane = lax.iota(jnp.int32, 16)

    def filt(src_hbm, role_off, off0, cap):
        def chunk(ci, off):
            pltpu.sync_copy(src_hbm.at[pl.ds(ci * 1024, 1024)], idb_v)

            def vreg(v, off):
                ids = idb_v[pl.ds(v * 16, 16)]
                m = (ids >= lo) & (ids < hib)
                slots = role_off + ci * 1024 + v * 16 + lane
                plsc.store_compressed(wli_v.at[pl.ds(off, 16)], ids, mask=m)
                plsc.store_compressed(wls_v.at[pl.ds(off, 16)], slots, mask=m)
                cnt = plsc.all_reduce_population_count(m)[0]
                return jnp.minimum(off + cnt, cap)

            return lax.fori_loop(0, 64, vreg, off)

        return lax.fori_loop(0, _BATCH // 1024, chunk, off0)

    def scan(table_hbm, tail_hbm, n, stage, trash_base):
        nb_wl = (n + 15) // 16

        def extract_window(a, nw, gather_fn):
            def batch_e(b, carry):
                ids = wwi_v[pl.ds(b * 16, 16)]
                sl = wws_v[pl.ds(b * 16, 16)]
                valid = (b * 16 + lane) < nw
                offv = jnp.where(valid, ids - a, 0)
                slv = jnp.where(valid, sl, trash_base + lane)
                for d in range(_EMB):
                    vals = gather_fn(d, offv)
                    plsc.store_scatter(
                        ex_v, [lane * _PITCH + jnp.int32(d)], vals)
                for j in range(16):
                    pltpu.async_copy(
                        ex_v.at[pl.ds(j * _PITCH, _EMB)],
                        stage.at[pl.ds(slv[j] * _EMB, _EMB)], semo)
                pltpu.make_async_copy(
                    stage.at[pl.ds(0, 16 * _EMB)],
                    ex_v.at[pl.ds(0, 16 * _EMB)], semo).wait()
                return carry

            lax.fori_loop(0, (nw + 15) // 16, batch_e, 0)

        def filter_window(cond_fn):
            def batch_f(b, wo):
                ids = wli_v[pl.ds(b * 16, 16)]
                sl = wls_v[pl.ds(b * 16, 16)]
                m = ((b * 16 + lane) < n) & cond_fn(ids)
                plsc.store_compressed(wwi_v.at[pl.ds(wo, 16)], ids, mask=m)
                plsc.store_compressed(wws_v.at[pl.ds(wo, 16)], sl, mask=m)
                cnt = plsc.all_reduce_population_count(m)[0]
                return jnp.minimum(wo + cnt, _WW_CAP)

            return lax.fori_loop(0, nb_wl, batch_f, 0)

        nk = jnp.maximum(
            0, jnp.minimum(_RW // _WIN, (_ALIGNED_TOP - lo) // _WIN))

        def window(k, carry):
            a = pl.multiple_of(lo + k * _WIN, 128)
            nw = filter_window(lambda ids: ((ids - lo) >> 10) == k)
            pltpu.async_copy(table_hbm.at[:, pl.ds(a, _WIN)], wb_v,
                             semw).wait()
            extract_window(
                a, nw,
                lambda d, offv: plsc.load_gather(
                    wb_v, [jnp.full((16,), d, jnp.int32), offv]))
            return carry

        lax.fori_loop(0, nk, window, 0)

        @pl.when((lo <= _ALIGNED_TOP) & (_ALIGNED_TOP < hib))
        def _tail():
            pltpu.sync_copy(tail_hbm, tt_v)
            nw = filter_window(lambda ids: ids >= _ALIGNED_TOP)
            extract_window(
                _ALIGNED_TOP, nw,
                lambda d, offv: plsc.load_gather(
                    tt_v, [offv * _EMB + jnp.int32(d)]))

    n_ent = filt(hi_hbm, 0, 0, _ENT_CAP)
    n_ent = filt(ti_hbm, _BATCH, n_ent, _ENT_CAP)
    scan(ent_hbm, etail_hbm, n_ent, ht_out, _HT_TRASH)
    n_rel = filt(ri_hbm, 0, 0, _REL_CAP)
    scan(rel_hbm, rtail_hbm, n_rel, r_out, _R_TRASH)


@functools.partial(
    pl.kernel,
    out_type=(
        jax.ShapeDtypeStruct(((2 * _BATCH + 16) * _EMB,), jnp.float32),
        jax.ShapeDtypeStruct(((_BATCH + 16) * _EMB,), jnp.float32),
    ),
    mesh=plsc.VectorSubcoreMesh(core_axis_name="c", subcore_axis_name="s"),
    compiler_params=pltpu.CompilerParams(needs_layout_passes=False),
    scratch_types=[
        pltpu.VMEM((1024,), jnp.int32),
        pltpu.VMEM((_ENT_CAP + 16,), jnp.int32),
        pltpu.VMEM((_ENT_CAP + 16,), jnp.int32),
        pltpu.VMEM((_WW_CAP + 16,), jnp.int32),
        pltpu.VMEM((_WW_CAP + 16,), jnp.int32),
        pltpu.VMEM((_EMB, _WIN), jnp.float32),
        pltpu.VMEM(((_ROWS - _ALIGNED_TOP) * _EMB,), jnp.float32),
        pltpu.VMEM((16 * _PITCH,), jnp.float32),
        pltpu.SemaphoreType.DMA,
        pltpu.SemaphoreType.DMA,
    ],
)
def _scan_sc(*refs):
    _scan_body(*refs)


def _math_body(ht_hbm, r_hbm, out_hbm, h_v, r_v, t_v, o_v, nrm_v):
    wid = lax.axis_index("s") * _NC + lax.axis_index("c")
    base = wid * _BPW

    pltpu.sync_copy(ht_hbm.at[pl.ds(base * _EMB, _BPW * _EMB)], h_v)
    pltpu.sync_copy(ht_hbm.at[pl.ds((_BATCH + base) * _EMB, _BPW * _EMB)],
                    t_v)
    pltpu.sync_copy(r_hbm.at[pl.ds(base * _EMB, _BPW * _EMB)], r_v)

    eps = jnp.float32(1e-24)
    zeros = jnp.zeros((16,), jnp.float32)
    ones_i = jnp.full((16,), 1, jnp.int32)
    c1 = jnp.full((16,), _BPW, jnp.int32)
    c2 = jnp.full((16,), 2 * _BPW, jnp.int32)

    def zero_nrm(b, carry):
        nrm_v[pl.ds(b * 16, 16)] = zeros
        return carry

    lax.fori_loop(0, _GROUPS * 3, zero_nrm, 0)

    def ssq_group(g, carry):
        jv = jnp.full((16,), g * 16, jnp.int32)
        for j in range(16):
            i = g * 16 + j
            hc = [h_v[pl.ds(i * _EMB + k * 16, 16)] for k in range(4)]
            rc = [r_v[pl.ds(i * _EMB + k * 16, 16)] for k in range(4)]
            tc = [t_v[pl.ds(i * _EMB + k * 16, 16)] for k in range(4)]
            sh = hc[0] * hc[0] + hc[1] * hc[1] + hc[2] * hc[2] + hc[3] * hc[3]
            sr = rc[0] * rc[0] + rc[1] * rc[1] + rc[2] * rc[2] + rc[3] * rc[3]
            st = tc[0] * tc[0] + tc[1] * tc[1] + tc[2] * tc[2] + tc[3] * tc[3]
            plsc.addupdate_scatter(nrm_v, [jv], sh)
            plsc.addupdate_scatter(nrm_v, [jv + c1], sr)
            plsc.addupdate_scatter(nrm_v, [jv + c2], st)
            jv = jv + ones_i
        return carry

    lax.fori_loop(0, _GROUPS, ssq_group, 0)

    def newton16(b, carry):
        nrm_v[pl.ds(b * 16, 16)] = _rsqrt_newton(
            jnp.maximum(nrm_v[pl.ds(b * 16, 16)], eps))
        return carry

    lax.fori_loop(0, _GROUPS * 3, newton16, 0)

    def zero_out(b, carry):
        o_v[pl.ds(b * 16, 16)] = zeros
        return carry

    lax.fori_loop(0, _GROUPS, zero_out, 0)

    def dist_group(g, carry):
        jv = jnp.full((16,), g * 16, jnp.int32)
        for j in range(16):
            i = g * 16 + j
            hc = [h_v[pl.ds(i * _EMB + k * 16, 16)] for k in range(4)]
            rc = [r_v[pl.ds(i * _EMB + k * 16, 16)] for k in range(4)]
            tc = [t_v[pl.ds(i * _EMB + k * 16, 16)] for k in range(4)]
            ih = plsc.load_gather(nrm_v, [jv])
            ir = plsc.load_gather(nrm_v, [jv + c1])
            it = plsc.load_gather(nrm_v, [jv + c2])
            s = jnp.abs(hc[0] * ih + rc[0] * ir - tc[0] * it)
            for k in range(1, 4):
                s = s + jnp.abs(hc[k] * ih + rc[k] * ir - tc[k] * it)
            plsc.addupdate_scatter(o_v, [jv], s)
            jv = jv + ones_i
        return carry

    lax.fori_loop(0, _GROUPS, dist_group, 0)
    pltpu.sync_copy(o_v, out_hbm.at[pl.ds(base, _BPW)])


@functools.partial(
    pl.kernel,
    out_type=jax.ShapeDtypeStruct((_BATCH,), jnp.float32),
    mesh=plsc.VectorSubcoreMesh(core_axis_name="c", subcore_axis_name="s"),
    compiler_params=pltpu.CompilerParams(needs_layout_passes=False),
    scratch_types=[
        pltpu.VMEM((_BPW * _EMB,), jnp.float32),
        pltpu.VMEM((_BPW * _EMB,), jnp.float32),
        pltpu.VMEM((_BPW * _EMB,), jnp.float32),
        pltpu.VMEM((_BPW,), jnp.float32),
        pltpu.VMEM((3 * _BPW,), jnp.float32),
    ],
)
def _math_sc(*refs):
    _math_body(*refs)


def kernel(edge, entity_embedding, relation_embedding):
    h_idx = edge[:, 0]
    r_idx = edge[:, 1]
    t_idx = edge[:, 2]
    ht_stage, r_stage = _scan_sc(h_idx, r_idx, t_idx,
                                 entity_embedding.T, relation_embedding.T,
                                 entity_embedding[_ALIGNED_TOP:].reshape(-1),
                                 relation_embedding[_ALIGNED_TOP:].reshape(-1))
    return _math_sc(ht_stage, r_stage)
